# baseline (speedup 1.0000x reference)
import jax
import jax.numpy as jnp
from jax.experimental import pallas as pl
from jax.experimental.pallas import tpu as pltpu

_N_LAYERS = 17


def _elu(x):
    return jnp.where(x > 0, x, jnp.exp(jnp.minimum(x, 0.0)) - 1.0)


def _mesh_encoder_body(pos_ref, adj_ref, *refs):
    w_refs = refs[:_N_LAYERS]
    b_refs = refs[_N_LAYERS:2 * _N_LAYERS]
    out_ref = refs[2 * _N_LAYERS]
    adjn_ref = refs[2 * _N_LAYERS + 1]

    adj = adj_ref[...]
    norm = jnp.sum(adj, axis=1, keepdims=True)
    inv_norm_row = (1.0 / norm).reshape(1, -1)
    adjn_ref[...] = adj * inv_norm_row
    adjn = adjn_ref[...]

    x = pos_ref[...]
    for i in range(_N_LAYERS):
        w = w_refs[i][...]
        b = b_refs[i][...]
        support = jnp.dot(x, w, preferred_element_type=jnp.float32,
                          precision=jax.lax.Precision.DEFAULT)
        side = max(support.shape[1] // 3, 2)
        side1 = jnp.dot(adjn, support[:, :side],
                        preferred_element_type=jnp.float32,
                        precision=jax.lax.Precision.DEFAULT)
        support = jnp.concatenate([side1, support[:, side:]], axis=1) + b
        if i < _N_LAYERS - 1:
            x = _elu(support)
        else:
            out_ref[...] = _elu(jnp.max(support, axis=0, keepdims=True))


def kernel(positions, adj, W0, W1, W2, W3, W4, W5, W6, W7, W8, W9, W10, W11, W12, W13, W14, W15, W16, b0, b1, b2, b3, b4, b5, b6, b7, b8, b9, b10, b11, b12, b13, b14, b15, b16):
    ws = [W0, W1, W2, W3, W4, W5, W6, W7, W8, W9, W10, W11, W12, W13, W14, W15, W16]
    bs = [b0, b1, b2, b3, b4, b5, b6, b7, b8, b9, b10, b11, b12, b13, b14, b15, b16]
    bs2d = [b.reshape(1, -1) for b in bs]
    n = adj.shape[0]
    out = pl.pallas_call(
        _mesh_encoder_body,
        out_shape=jax.ShapeDtypeStruct((1, ws[-1].shape[1]), jnp.float32),
        scratch_shapes=[pltpu.VMEM((n, n), jnp.float32)],
        compiler_params=pltpu.CompilerParams(
            vmem_limit_bytes=100 * 1024 * 1024,
        ),
    )(positions, adj, *ws, *bs2d)
    return out.reshape(-1)


# speedup vs baseline: 1.0091x; 1.0091x over previous
import jax
import jax.numpy as jnp
from jax.experimental import pallas as pl
from jax.experimental.pallas import tpu as pltpu

_N_LAYERS = 17
_N = 2562
_SPLITS = (0, 1280, _N)


def _elu(x):
    return jnp.where(x > 0, x, jnp.exp(jnp.minimum(x, 0.0)) - 1.0)


def _dot(a, b):
    return jnp.dot(a, b, preferred_element_type=jnp.float32,
                   precision=jax.lax.Precision.DEFAULT)


def _mesh_encoder_body(pos_ref, adj_ref, *refs):
    w_refs = refs[:_N_LAYERS]
    b_refs = refs[_N_LAYERS:2 * _N_LAYERS]
    out_ref = refs[2 * _N_LAYERS]

    adj = adj_ref[...]
    norm = jnp.sum(adj, axis=1, keepdims=True)
    nchunks = len(_SPLITS) - 1
    bounds = list(zip(_SPLITS[:-1], _SPLITS[1:]))
    inv_norm = [1.0 / norm[lo:hi] for lo, hi in bounds]
    adj_cols = [adj[:, lo:hi] for lo, hi in bounds]

    xs = [pos_ref[lo:hi, :] for lo, hi in bounds]
    for i in range(_N_LAYERS):
        w = w_refs[i][...]
        b = b_refs[i][...]
        ss = [_dot(xs[c], w) for c in range(nchunks)]
        side = max(w.shape[1] // 3, 2)
        p = _dot(adj_cols[0], ss[0][:, :side] * inv_norm[0])
        for c in range(1, nchunks):
            p = p + _dot(adj_cols[c], ss[c][:, :side] * inv_norm[c])
        sup = [
            jnp.concatenate([p[lo:hi], ss[c][:, side:]], axis=1) + b
            for c, (lo, hi) in enumerate(bounds)
        ]
        if i < _N_LAYERS - 1:
            xs = [_elu(s) for s in sup]
        else:
            m = jnp.max(sup[0], axis=0, keepdims=True)
            for c in range(1, nchunks):
                m = jnp.maximum(m, jnp.max(sup[c], axis=0, keepdims=True))
            out_ref[...] = _elu(m)


def kernel(positions, adj, W0, W1, W2, W3, W4, W5, W6, W7, W8, W9, W10, W11, W12, W13, W14, W15, W16, b0, b1, b2, b3, b4, b5, b6, b7, b8, b9, b10, b11, b12, b13, b14, b15, b16):
    ws = [W0, W1, W2, W3, W4, W5, W6, W7, W8, W9, W10, W11, W12, W13, W14, W15, W16]
    bs = [b0, b1, b2, b3, b4, b5, b6, b7, b8, b9, b10, b11, b12, b13, b14, b15, b16]
    bs2d = [b.reshape(1, -1) for b in bs]
    out = pl.pallas_call(
        _mesh_encoder_body,
        out_shape=jax.ShapeDtypeStruct((1, ws[-1].shape[1]), jnp.float32),
        compiler_params=pltpu.CompilerParams(
            vmem_limit_bytes=100 * 1024 * 1024,
        ),
    )(positions, adj, *ws, *bs2d)
    return out.reshape(-1)


# speedup vs baseline: 1.6006x; 1.5862x over previous
import jax
import jax.numpy as jnp
from jax.experimental import pallas as pl
from jax.experimental.pallas import tpu as pltpu

_N_LAYERS = 17
_N = 2562
_SPLITS = (0, 1280, _N)


def _elu(x):
    return jnp.where(x > 0, x, jnp.exp(jnp.minimum(x, 0.0)) - 1.0)


def _dot(a, b):
    return jnp.dot(a, b, preferred_element_type=jnp.float32,
                   precision=jax.lax.Precision.DEFAULT)


def _mesh_encoder_body(pos_ref, adj_ref, *refs):
    w_refs = refs[:_N_LAYERS]
    b_refs = refs[_N_LAYERS:2 * _N_LAYERS]
    out_ref = refs[2 * _N_LAYERS]

    adj = adj_ref[...]
    norm = jnp.sum(adj, axis=1, keepdims=True)
    nchunks = len(_SPLITS) - 1
    bounds = list(zip(_SPLITS[:-1], _SPLITS[1:]))
    inv_norm = [1.0 / norm[lo:hi] for lo, hi in bounds]
    adj_rows = [adj[lo:hi, :] for lo, hi in bounds]

    xs = [pos_ref[lo:hi, :] for lo, hi in bounds]
    for i in range(_N_LAYERS):
        w = w_refs[i][...]
        b = b_refs[i][...]
        ss = [_dot(xs[c], w) for c in range(nchunks)]
        side = max(w.shape[1] // 3, 2)
        ns = jnp.concatenate(
            [ss[c][:, :side] * inv_norm[c] for c in range(nchunks)], axis=0)
        ps = [_dot(adj_rows[c], ns) for c in range(nchunks)]
        sup = [
            jnp.concatenate([ps[c], ss[c][:, side:]], axis=1) + b
            for c in range(nchunks)
        ]
        if i < _N_LAYERS - 1:
            xs = [_elu(s) for s in sup]
        else:
            m = jnp.max(sup[0], axis=0, keepdims=True)
            for c in range(1, nchunks):
                m = jnp.maximum(m, jnp.max(sup[c], axis=0, keepdims=True))
            out_ref[...] = _elu(m)


def kernel(positions, adj, W0, W1, W2, W3, W4, W5, W6, W7, W8, W9, W10, W11, W12, W13, W14, W15, W16, b0, b1, b2, b3, b4, b5, b6, b7, b8, b9, b10, b11, b12, b13, b14, b15, b16):
    ws = [W0, W1, W2, W3, W4, W5, W6, W7, W8, W9, W10, W11, W12, W13, W14, W15, W16]
    bs = [b0, b1, b2, b3, b4, b5, b6, b7, b8, b9, b10, b11, b12, b13, b14, b15, b16]
    bs2d = [b.reshape(1, -1) for b in bs]
    out = pl.pallas_call(
        _mesh_encoder_body,
        out_shape=jax.ShapeDtypeStruct((1, ws[-1].shape[1]), jnp.float32),
        compiler_params=pltpu.CompilerParams(
            vmem_limit_bytes=100 * 1024 * 1024,
        ),
    )(positions, adj, *ws, *bs2d)
    return out.reshape(-1)


# speedup vs baseline: 1.6381x; 1.0234x over previous
import jax
import jax.numpy as jnp
from jax.experimental import pallas as pl
from jax.experimental.pallas import tpu as pltpu

_N_LAYERS = 17
_N = 2562
_SPLITS = (0, 640, 1280, 1920, _N)


def _elu(x):
    return jnp.where(x > 0, x, jnp.exp(jnp.minimum(x, 0.0)) - 1.0)


def _dot(a, b):
    return jnp.dot(a, b, preferred_element_type=jnp.float32,
                   precision=jax.lax.Precision.DEFAULT)


def _mesh_encoder_body(pos_ref, adj_ref, *refs):
    w_refs = refs[:_N_LAYERS]
    b_refs = refs[_N_LAYERS:2 * _N_LAYERS]
    out_ref = refs[2 * _N_LAYERS]

    adj = adj_ref[...]
    norm = jnp.sum(adj, axis=1, keepdims=True)
    nchunks = len(_SPLITS) - 1
    bounds = list(zip(_SPLITS[:-1], _SPLITS[1:]))
    inv_norm = [1.0 / norm[lo:hi] for lo, hi in bounds]
    adj_rows = [adj[lo:hi, :] for lo, hi in bounds]

    xs = [pos_ref[lo:hi, :] for lo, hi in bounds]
    for i in range(_N_LAYERS):
        w = w_refs[i][...]
        b = b_refs[i][...]
        ss = [_dot(xs[c], w) for c in range(nchunks)]
        side = max(w.shape[1] // 3, 2)
        ns = jnp.concatenate(
            [ss[c][:, :side] * inv_norm[c] for c in range(nchunks)], axis=0)
        ps = [_dot(adj_rows[c], ns) for c in range(nchunks)]
        sup = [
            jnp.concatenate([ps[c], ss[c][:, side:]], axis=1) + b
            for c in range(nchunks)
        ]
        if i < _N_LAYERS - 1:
            xs = [_elu(s) for s in sup]
        else:
            m = jnp.max(sup[0], axis=0, keepdims=True)
            for c in range(1, nchunks):
                m = jnp.maximum(m, jnp.max(sup[c], axis=0, keepdims=True))
            out_ref[...] = _elu(m)


def kernel(positions, adj, W0, W1, W2, W3, W4, W5, W6, W7, W8, W9, W10, W11, W12, W13, W14, W15, W16, b0, b1, b2, b3, b4, b5, b6, b7, b8, b9, b10, b11, b12, b13, b14, b15, b16):
    ws = [W0, W1, W2, W3, W4, W5, W6, W7, W8, W9, W10, W11, W12, W13, W14, W15, W16]
    bs = [b0, b1, b2, b3, b4, b5, b6, b7, b8, b9, b10, b11, b12, b13, b14, b15, b16]
    bs2d = [b.reshape(1, -1) for b in bs]
    out = pl.pallas_call(
        _mesh_encoder_body,
        out_shape=jax.ShapeDtypeStruct((1, ws[-1].shape[1]), jnp.float32),
        compiler_params=pltpu.CompilerParams(
            vmem_limit_bytes=100 * 1024 * 1024,
        ),
    )(positions, adj, *ws, *bs2d)
    return out.reshape(-1)
